# baseline (speedup 1.0000x reference)
import functools

import jax
import jax.numpy as jnp
from jax import lax
from jax.experimental import pallas as pl
from jax.experimental.pallas import tpu as pltpu
from jax.experimental.pallas import tpu_sc as plsc

_NC = 2
_NS = 16
_NW = _NC * _NS
_CHUNK = 64

_BS = 512


def _sc_gather(idx, emb):
    S = idx.shape[0]
    D = emb.shape[1]
    rows_per_w = S // _NW
    n_chunks = rows_per_w // _CHUNK
    mesh = plsc.VectorSubcoreMesh(core_axis_name="c", subcore_axis_name="s")

    @functools.partial(
        pl.kernel,
        out_type=jax.ShapeDtypeStruct((S, D), jnp.float32),
        mesh=mesh,
        scratch_types=[
            pltpu.VMEM((rows_per_w,), jnp.int32),
            pltpu.VMEM((_CHUNK, D), jnp.float32),
            pltpu.SemaphoreType.DMA,
        ],
    )
    def gather_kernel(idx_hbm, emb_hbm, pe_hbm, idx_v, rows_v, sem):
        wid = lax.axis_index("s") * _NC + lax.axis_index("c")
        base = wid * rows_per_w
        pltpu.sync_copy(idx_hbm.at[pl.ds(base, rows_per_w)], idx_v)
        for c in range(n_chunks):
            idx_slice = idx_v.at[pl.ds(c * _CHUNK, _CHUNK)]
            pltpu.async_copy(emb_hbm.at[idx_slice], rows_v, sem).wait()
            pltpu.sync_copy(rows_v, pe_hbm.at[pl.ds(base + c * _CHUNK, _CHUNK)])

    return gather_kernel(idx, emb)


def _add_body(x_ref, pe_ref, out_ref):
    out_ref[...] = x_ref[...] + pe_ref[...][None, :, :]


def _tc_add(x, pe):
    B, S, D = x.shape
    return pl.pallas_call(
        _add_body,
        grid=(S // _BS,),
        in_specs=[
            pl.BlockSpec((B, _BS, D), lambda i: (0, i, 0)),
            pl.BlockSpec((_BS, D), lambda i: (i, 0)),
        ],
        out_specs=pl.BlockSpec((B, _BS, D), lambda i: (0, i, 0)),
        out_shape=jax.ShapeDtypeStruct((B, S, D), x.dtype),
    )(x, pe)


def kernel(x, pos_ids, emb):
    B, S, D = x.shape
    idx = pos_ids[0, :S].astype(jnp.int32)
    pe = _sc_gather(idx, emb)
    return _tc_add(x, pe)
